# baseline (speedup 1.0000x reference)
import functools

import jax
import jax.numpy as jnp
from jax import lax
from jax.experimental import pallas as pl
from jax.experimental.pallas import tpu as pltpu
from jax.experimental.pallas import tpu_sc as plsc

N_NODES = 100000
BATCH = 10000
DEG = 32
D = 128
G = 4
IDX_PER_G = G * DEG
NGROUPS = BATCH // G
NBUF = 4


@functools.lru_cache(maxsize=1)
def _build():
    info = plsc.get_sparse_core_info()
    NC, NS, L = info.num_cores, info.num_subcores, info.num_lanes
    NW = NC * NS
    NV = D // L
    KLO = NGROUPS // NW
    NHI = NGROUPS - KLO * NW
    K = KLO + 1
    K_PAD = -(-K // NBUF) * NBUF

    mesh = plsc.VectorSubcoreMesh(core_axis_name="c", subcore_axis_name="s")

    scratch = [pltpu.VMEM((K * IDX_PER_G,), jnp.int32)]
    for _ in range(NBUF):
        scratch += [
            pltpu.VMEM((IDX_PER_G, D), jnp.float32),
            pltpu.VMEM((G, D), jnp.float32),
            pltpu.SemaphoreType.DMA,
            pltpu.SemaphoreType.DMA,
        ]
    scratch.append(pltpu.VMEM((D,), jnp.float32))

    @functools.partial(
        pl.kernel,
        mesh=mesh,
        out_type=jax.ShapeDtypeStruct((BATCH, D), jnp.float32),
        scratch_types=scratch,
        compiler_params=pltpu.CompilerParams(needs_layout_passes=False),
    )
    def agg(table_hbm, neigh_hbm, out_hbm, idx_all, *refs):
        bufs = [refs[4 * b:4 * b + 4] for b in range(NBUF)]
        row0_v = refs[4 * NBUF]
        wid = lax.axis_index("s") * NC + lax.axis_index("c")
        kw = jnp.where(wid < NHI, KLO + 1, KLO)
        g0 = wid * KLO + jnp.minimum(wid, NHI)
        gs = jnp.minimum(g0, NGROUPS - K)
        ofs = (g0 - gs) * IDX_PER_G

        pltpu.sync_copy(table_hbm.at[0], row0_v)
        pltpu.sync_copy(
            neigh_hbm.at[pl.ds(gs * IDX_PER_G, K * IDX_PER_G)], idx_all)

        def start_fetch(j, rows_v, sem):
            @pl.when(j < kw)
            def _():
                pltpu.async_copy(
                    table_hbm.at[
                        idx_all.at[pl.ds(ofs + j * IDX_PER_G, IDX_PER_G)]],
                    rows_v, sem)

        for b in range(NBUF):
            start_fetch(b, bufs[b][0], bufs[b][2])

        def step(i, carry):
            for b in range(NBUF):
                rows_v, out_v, sem, sem_o = bufs[b]
                j = i * NBUF + b

                @pl.when(j < kw)
                def _(rows_v=rows_v, out_v=out_v, sem=sem, sem_o=sem_o, j=j):
                    pltpu.make_async_copy(
                        table_hbm.at[idx_all.at[pl.ds(ofs + j * IDX_PER_G,
                                                      IDX_PER_G)]],
                        rows_v, sem).wait()

                    @pl.when(j >= NBUF)
                    def _():
                        pltpu.make_async_copy(
                            out_v, out_hbm.at[pl.ds(0, G)], sem_o).wait()

                    for r in range(G):
                        i0 = idx_all[pl.ds(ofs + j * IDX_PER_G + r * DEG, L)]
                        i1 = idx_all[pl.ds(ofs + j * IDX_PER_G + r * DEG + L,
                                           L)]
                        nz_s = jnp.sum(jnp.where(i0 == 0, 1.0, 0.0)
                                       + jnp.where(i1 == 0, 1.0, 0.0))
                        nzf = jnp.full((L,), nz_s, dtype=jnp.float32)

                        acc = tuple(jnp.zeros((L,), jnp.float32)
                                    for _ in range(NV))
                        for n in range(DEG):
                            acc = tuple(
                                acc[v] + rows_v[r * DEG + n, pl.ds(v * L, L)]
                                for v in range(NV))
                        cnt = jnp.float32(DEG) - nzf
                        cnt = jnp.where(cnt == 0.0, 1.0, cnt)
                        scale = 1.0 / cnt
                        for v in range(NV):
                            out_v[r, pl.ds(v * L, L)] = (
                                acc[v] - nzf * row0_v[pl.ds(v * L, L)]) * scale

                    pltpu.async_copy(
                        out_v, out_hbm.at[pl.ds((g0 + j) * G, G)], sem_o)
                    start_fetch(j + NBUF, rows_v, sem)

            return carry

        lax.fori_loop(0, K_PAD // NBUF, step, 0)

        for b in range(NBUF):
            pltpu.make_async_copy(
                bufs[b][1], out_hbm.at[pl.ds(0, G)], bufs[b][3]).wait()

    return agg


def kernel(feature_table, nodes, neigh_index, feature_dim):
    del nodes, feature_dim
    neigh_flat = neigh_index.reshape(-1).astype(jnp.int32)
    return _build()(feature_table, neigh_flat)


# speedup vs baseline: 2.6702x; 2.6702x over previous
import functools

import jax
import jax.numpy as jnp
from jax import lax
from jax.experimental import pallas as pl
from jax.experimental.pallas import tpu as pltpu
from jax.experimental.pallas import tpu_sc as plsc

N_NODES = 100000
BATCH = 10000
DEG = 32
D = 128
G = 4
IDX_PER_G = G * DEG
NGROUPS = BATCH // G
NBUF = 4


@functools.lru_cache(maxsize=1)
def _build():
    info = plsc.get_sparse_core_info()
    NC, NS, L = info.num_cores, info.num_subcores, info.num_lanes
    NW = NC * NS
    NV = D // L
    KLO = NGROUPS // NW
    NHI = NGROUPS - KLO * NW
    K = KLO + 1
    K_PAD = -(-K // NBUF) * NBUF

    mesh = plsc.VectorSubcoreMesh(core_axis_name="c", subcore_axis_name="s")

    scratch = [pltpu.VMEM((K * IDX_PER_G,), jnp.int32)]
    for _ in range(NBUF):
        scratch += [
            pltpu.VMEM((IDX_PER_G, D), jnp.float32),
            pltpu.VMEM((G, D), jnp.float32),
            pltpu.SemaphoreType.DMA,
            pltpu.SemaphoreType.DMA,
        ]
    scratch.append(pltpu.VMEM((D,), jnp.float32))

    @functools.partial(
        pl.kernel,
        mesh=mesh,
        out_type=jax.ShapeDtypeStruct((BATCH, D), jnp.float32),
        scratch_types=scratch,
        compiler_params=pltpu.CompilerParams(needs_layout_passes=False),
    )
    def agg(table_hbm, neigh_hbm, out_hbm, idx_all, *refs):
        bufs = [refs[4 * b:4 * b + 4] for b in range(NBUF)]
        row0_v = refs[4 * NBUF]
        wid = lax.axis_index("s") * NC + lax.axis_index("c")
        kw = jnp.where(wid < NHI, KLO + 1, KLO)
        g0 = wid * KLO + jnp.minimum(wid, NHI)
        gs = jnp.minimum(g0, NGROUPS - K)
        ofs = (g0 - gs) * IDX_PER_G

        pltpu.sync_copy(table_hbm.at[0], row0_v)
        pltpu.sync_copy(
            neigh_hbm.at[pl.ds(gs * IDX_PER_G, K * IDX_PER_G)], idx_all)

        def start_fetch(j, rows_v, sem):
            @pl.when(j < kw)
            def _():
                pltpu.async_copy(
                    table_hbm.at[
                        idx_all.at[pl.ds(ofs + j * IDX_PER_G, IDX_PER_G)]],
                    rows_v, sem)

        for b in range(NBUF):
            start_fetch(b, bufs[b][0], bufs[b][2])

        def step(i, carry):
            for b in range(NBUF):
                rows_v, out_v, sem, sem_o = bufs[b]
                j = i * NBUF + b

                @pl.when(j < kw)
                def _(rows_v=rows_v, out_v=out_v, sem=sem, sem_o=sem_o, j=j):
                    pltpu.make_async_copy(
                        table_hbm.at[idx_all.at[pl.ds(ofs + j * IDX_PER_G,
                                                      IDX_PER_G)]],
                        rows_v, sem).wait()

                    @pl.when(j >= NBUF)
                    def _():
                        pltpu.make_async_copy(
                            out_v, out_hbm.at[pl.ds(0, G)], sem_o).wait()

                    for r in range(G):
                        i0 = idx_all[pl.ds(ofs + j * IDX_PER_G + r * DEG, L)]
                        i1 = idx_all[pl.ds(ofs + j * IDX_PER_G + r * DEG + L,
                                           L)]
                        nz_s = jnp.sum(jnp.where(i0 == 0, 1.0, 0.0)
                                       + jnp.where(i1 == 0, 1.0, 0.0))
                        nzf = jnp.full((L,), nz_s, dtype=jnp.float32)

                        def body(n4, acc):
                            row = r * DEG + 4 * n4
                            for u in range(4):
                                acc = tuple(
                                    acc[v] + rows_v[row + u, pl.ds(v * L, L)]
                                    for v in range(NV))
                            return acc

                        acc = lax.fori_loop(
                            0, DEG // 4, body,
                            tuple(jnp.zeros((L,), jnp.float32)
                                  for _ in range(NV)))
                        cnt = jnp.float32(DEG) - nzf
                        cnt = jnp.where(cnt == 0.0, 1.0, cnt)
                        scale = 1.0 / cnt
                        for v in range(NV):
                            out_v[r, pl.ds(v * L, L)] = (
                                acc[v] - nzf * row0_v[pl.ds(v * L, L)]) * scale

                    pltpu.async_copy(
                        out_v, out_hbm.at[pl.ds((g0 + j) * G, G)], sem_o)
                    start_fetch(j + NBUF, rows_v, sem)

            return carry

        lax.fori_loop(0, K_PAD // NBUF, step, 0)

        for b in range(NBUF):
            pltpu.make_async_copy(
                bufs[b][1], out_hbm.at[pl.ds(0, G)], bufs[b][3]).wait()

    return agg


def kernel(feature_table, nodes, neigh_index, feature_dim):
    del nodes, feature_dim
    neigh_flat = neigh_index.reshape(-1).astype(jnp.int32)
    return _build()(feature_table, neigh_flat)


# speedup vs baseline: 2.6968x; 1.0100x over previous
import functools

import jax
import jax.numpy as jnp
from jax import lax
from jax.experimental import pallas as pl
from jax.experimental.pallas import tpu as pltpu
from jax.experimental.pallas import tpu_sc as plsc

N_NODES = 100000
BATCH = 10000
DEG = 32
D = 128
G = 4
IDX_PER_G = G * DEG
NGROUPS = BATCH // G
NBUF = 4


@functools.lru_cache(maxsize=1)
def _build():
    info = plsc.get_sparse_core_info()
    NC, NS, L = info.num_cores, info.num_subcores, info.num_lanes
    NW = NC * NS
    NV = D // L
    KLO = NGROUPS // NW
    NHI = NGROUPS - KLO * NW
    K = KLO + 1
    K_PAD = -(-K // NBUF) * NBUF

    mesh = plsc.VectorSubcoreMesh(core_axis_name="c", subcore_axis_name="s")

    scratch = [pltpu.VMEM((K * IDX_PER_G,), jnp.int32)]
    for _ in range(NBUF):
        scratch += [
            pltpu.VMEM((IDX_PER_G, D), jnp.float32),
            pltpu.VMEM((G, D), jnp.float32),
            pltpu.SemaphoreType.DMA,
            pltpu.SemaphoreType.DMA,
        ]
    scratch.append(pltpu.VMEM((D,), jnp.float32))

    @functools.partial(
        pl.kernel,
        mesh=mesh,
        out_type=jax.ShapeDtypeStruct((BATCH, D), jnp.float32),
        scratch_types=scratch,
        compiler_params=pltpu.CompilerParams(needs_layout_passes=False),
    )
    def agg(table_hbm, neigh_hbm, out_hbm, idx_all, *refs):
        bufs = [refs[4 * b:4 * b + 4] for b in range(NBUF)]
        row0_v = refs[4 * NBUF]
        wid = lax.axis_index("s") * NC + lax.axis_index("c")
        kw = jnp.where(wid < NHI, KLO + 1, KLO)
        g0 = wid * KLO + jnp.minimum(wid, NHI)
        gs = jnp.minimum(g0, NGROUPS - K)
        ofs = (g0 - gs) * IDX_PER_G

        pltpu.sync_copy(table_hbm.at[0], row0_v)
        pltpu.sync_copy(
            neigh_hbm.at[pl.ds(gs * IDX_PER_G, K * IDX_PER_G)], idx_all)

        def start_fetch(j, rows_v, sem):
            @pl.when(j < kw)
            def _():
                pltpu.async_copy(
                    table_hbm.at[
                        idx_all.at[pl.ds(ofs + j * IDX_PER_G, IDX_PER_G)]],
                    rows_v, sem)

        for b in range(NBUF):
            start_fetch(b, bufs[b][0], bufs[b][2])

        def step(i, carry):
            for b in range(NBUF):
                rows_v, out_v, sem, sem_o = bufs[b]
                j = i * NBUF + b

                @pl.when(j < kw)
                def _(rows_v=rows_v, out_v=out_v, sem=sem, sem_o=sem_o, j=j):
                    pltpu.make_async_copy(
                        table_hbm.at[idx_all.at[pl.ds(ofs + j * IDX_PER_G,
                                                      IDX_PER_G)]],
                        rows_v, sem).wait()

                    @pl.when(j >= NBUF)
                    def _():
                        pltpu.make_async_copy(
                            out_v, out_hbm.at[pl.ds(0, G)], sem_o).wait()

                    for r in range(G):
                        i0 = idx_all[pl.ds(ofs + j * IDX_PER_G + r * DEG, L)]
                        i1 = idx_all[pl.ds(ofs + j * IDX_PER_G + r * DEG + L,
                                           L)]
                        nz_s = jnp.sum(jnp.where(i0 == 0, 1.0, 0.0)
                                       + jnp.where(i1 == 0, 1.0, 0.0))
                        nzf = jnp.full((L,), nz_s, dtype=jnp.float32)

                        def body(n4, acc):
                            row = r * DEG + 2 * n4
                            for u in range(2):
                                acc = tuple(
                                    acc[v] + rows_v[row + u, pl.ds(v * L, L)]
                                    for v in range(NV))
                            return acc

                        acc = lax.fori_loop(
                            0, DEG // 2, body,
                            tuple(jnp.zeros((L,), jnp.float32)
                                  for _ in range(NV)))
                        cnt = jnp.float32(DEG) - nzf
                        cnt = jnp.where(cnt == 0.0, 1.0, cnt)
                        scale = 1.0 / cnt
                        for v in range(NV):
                            out_v[r, pl.ds(v * L, L)] = (
                                acc[v] - nzf * row0_v[pl.ds(v * L, L)]) * scale

                    pltpu.async_copy(
                        out_v, out_hbm.at[pl.ds((g0 + j) * G, G)], sem_o)
                    start_fetch(j + NBUF, rows_v, sem)

            return carry

        lax.fori_loop(0, K_PAD // NBUF, step, 0)

        for b in range(NBUF):
            pltpu.make_async_copy(
                bufs[b][1], out_hbm.at[pl.ds(0, G)], bufs[b][3]).wait()

    return agg


def kernel(feature_table, nodes, neigh_index, feature_dim):
    del nodes, feature_dim
    neigh_flat = neigh_index.reshape(-1).astype(jnp.int32)
    return _build()(feature_table, neigh_flat)


# speedup vs baseline: 2.7107x; 1.0051x over previous
import functools

import jax
import jax.numpy as jnp
from jax import lax
from jax.experimental import pallas as pl
from jax.experimental.pallas import tpu as pltpu
from jax.experimental.pallas import tpu_sc as plsc

N_NODES = 100000
BATCH = 10000
DEG = 32
D = 128
G = 4
IDX_PER_G = G * DEG
NGROUPS = BATCH // G
NBUF = 4


@functools.lru_cache(maxsize=1)
def _build():
    info = plsc.get_sparse_core_info()
    NC, NS, L = info.num_cores, info.num_subcores, info.num_lanes
    NW = NC * NS
    NV = D // L
    KLO = NGROUPS // NW
    NHI = NGROUPS - KLO * NW
    K = KLO + 1
    K_PAD = -(-K // NBUF) * NBUF

    mesh = plsc.VectorSubcoreMesh(core_axis_name="c", subcore_axis_name="s")

    scratch = [pltpu.VMEM((K * IDX_PER_G,), jnp.int32)]
    for _ in range(NBUF):
        scratch += [
            pltpu.VMEM((IDX_PER_G, D), jnp.float32),
            pltpu.VMEM((G, D), jnp.float32),
            pltpu.SemaphoreType.DMA,
            pltpu.SemaphoreType.DMA,
        ]
    scratch.append(pltpu.VMEM((D,), jnp.float32))

    @functools.partial(
        pl.kernel,
        mesh=mesh,
        out_type=jax.ShapeDtypeStruct((BATCH, D), jnp.float32),
        scratch_types=scratch,
        compiler_params=pltpu.CompilerParams(needs_layout_passes=False),
    )
    def agg(table_hbm, neigh_hbm, out_hbm, idx_all, *refs):
        bufs = [refs[4 * b:4 * b + 4] for b in range(NBUF)]
        row0_v = refs[4 * NBUF]
        wid = lax.axis_index("s") * NC + lax.axis_index("c")
        kw = jnp.where(wid < NHI, KLO + 1, KLO)
        g0 = wid * KLO + jnp.minimum(wid, NHI)
        gs = jnp.minimum(g0, NGROUPS - K)
        ofs = (g0 - gs) * IDX_PER_G

        pltpu.sync_copy(table_hbm.at[0], row0_v)
        pltpu.sync_copy(
            neigh_hbm.at[pl.ds(gs * IDX_PER_G, K * IDX_PER_G)], idx_all)

        def start_fetch(j, rows_v, sem):
            @pl.when(j < kw)
            def _():
                pltpu.async_copy(
                    table_hbm.at[
                        idx_all.at[pl.ds(ofs + j * IDX_PER_G, IDX_PER_G)]],
                    rows_v, sem)

        for b in range(NBUF):
            start_fetch(b, bufs[b][0], bufs[b][2])

        def step(i, carry):
            for b in range(NBUF):
                rows_v, out_v, sem, sem_o = bufs[b]
                j = i * NBUF + b

                @pl.when(j < kw)
                def _(rows_v=rows_v, out_v=out_v, sem=sem, sem_o=sem_o, j=j):
                    pltpu.make_async_copy(
                        table_hbm.at[idx_all.at[pl.ds(ofs + j * IDX_PER_G,
                                                      IDX_PER_G)]],
                        rows_v, sem).wait()

                    @pl.when(j >= NBUF)
                    def _():
                        pltpu.make_async_copy(
                            out_v, out_hbm.at[pl.ds(0, G)], sem_o).wait()

                    for r in range(G):
                        i0 = idx_all[pl.ds(ofs + j * IDX_PER_G + r * DEG, L)]
                        i1 = idx_all[pl.ds(ofs + j * IDX_PER_G + r * DEG + L,
                                           L)]
                        nz_s = jnp.sum(jnp.where(i0 == 0, 1.0, 0.0)
                                       + jnp.where(i1 == 0, 1.0, 0.0))
                        nzf = jnp.full((L,), nz_s, dtype=jnp.float32)

                        def body(n, acc):
                            row = r * DEG + n
                            return tuple(
                                acc[v] + rows_v[row, pl.ds(v * L, L)]
                                for v in range(NV))

                        acc = lax.fori_loop(
                            0, DEG, body,
                            tuple(jnp.zeros((L,), jnp.float32)
                                  for _ in range(NV)))
                        cnt = jnp.float32(DEG) - nzf
                        cnt = jnp.where(cnt == 0.0, 1.0, cnt)
                        scale = 1.0 / cnt
                        for v in range(NV):
                            out_v[r, pl.ds(v * L, L)] = (
                                acc[v] - nzf * row0_v[pl.ds(v * L, L)]) * scale

                    pltpu.async_copy(
                        out_v, out_hbm.at[pl.ds((g0 + j) * G, G)], sem_o)
                    start_fetch(j + NBUF, rows_v, sem)

            return carry

        lax.fori_loop(0, K_PAD // NBUF, step, 0)

        for b in range(NBUF):
            pltpu.make_async_copy(
                bufs[b][1], out_hbm.at[pl.ds(0, G)], bufs[b][3]).wait()

    return agg


def kernel(feature_table, nodes, neigh_index, feature_dim):
    del nodes, feature_dim
    neigh_flat = neigh_index.reshape(-1).astype(jnp.int32)
    return _build()(feature_table, neigh_flat)


# speedup vs baseline: 2.7131x; 1.0009x over previous
import functools

import jax
import jax.numpy as jnp
from jax import lax
from jax.experimental import pallas as pl
from jax.experimental.pallas import tpu as pltpu
from jax.experimental.pallas import tpu_sc as plsc

N_NODES = 100000
BATCH = 10000
DEG = 32
D = 128
G = 4
IDX_PER_G = G * DEG
NGROUPS = BATCH // G
NBUF = 5


@functools.lru_cache(maxsize=1)
def _build():
    info = plsc.get_sparse_core_info()
    NC, NS, L = info.num_cores, info.num_subcores, info.num_lanes
    NW = NC * NS
    NV = D // L
    KLO = NGROUPS // NW
    NHI = NGROUPS - KLO * NW
    K = KLO + 1
    K_PAD = -(-K // NBUF) * NBUF

    mesh = plsc.VectorSubcoreMesh(core_axis_name="c", subcore_axis_name="s")

    scratch = [pltpu.VMEM((K * IDX_PER_G,), jnp.int32)]
    for _ in range(NBUF):
        scratch += [
            pltpu.VMEM((IDX_PER_G, D), jnp.float32),
            pltpu.VMEM((G, D), jnp.float32),
            pltpu.SemaphoreType.DMA,
            pltpu.SemaphoreType.DMA,
        ]
    scratch.append(pltpu.VMEM((D,), jnp.float32))

    @functools.partial(
        pl.kernel,
        mesh=mesh,
        out_type=jax.ShapeDtypeStruct((BATCH, D), jnp.float32),
        scratch_types=scratch,
        compiler_params=pltpu.CompilerParams(needs_layout_passes=False),
    )
    def agg(table_hbm, neigh_hbm, out_hbm, idx_all, *refs):
        bufs = [refs[4 * b:4 * b + 4] for b in range(NBUF)]
        row0_v = refs[4 * NBUF]
        wid = lax.axis_index("s") * NC + lax.axis_index("c")
        kw = jnp.where(wid < NHI, KLO + 1, KLO)
        g0 = wid * KLO + jnp.minimum(wid, NHI)
        gs = jnp.minimum(g0, NGROUPS - K)
        ofs = (g0 - gs) * IDX_PER_G

        pltpu.sync_copy(table_hbm.at[0], row0_v)
        pltpu.sync_copy(
            neigh_hbm.at[pl.ds(gs * IDX_PER_G, K * IDX_PER_G)], idx_all)

        def start_fetch(j, rows_v, sem):
            @pl.when(j < kw)
            def _():
                pltpu.async_copy(
                    table_hbm.at[
                        idx_all.at[pl.ds(ofs + j * IDX_PER_G, IDX_PER_G)]],
                    rows_v, sem)

        for b in range(NBUF):
            start_fetch(b, bufs[b][0], bufs[b][2])

        def step(i, carry):
            for b in range(NBUF):
                rows_v, out_v, sem, sem_o = bufs[b]
                j = i * NBUF + b

                @pl.when(j < kw)
                def _(rows_v=rows_v, out_v=out_v, sem=sem, sem_o=sem_o, j=j):
                    pltpu.make_async_copy(
                        table_hbm.at[idx_all.at[pl.ds(ofs + j * IDX_PER_G,
                                                      IDX_PER_G)]],
                        rows_v, sem).wait()

                    @pl.when(j >= NBUF)
                    def _():
                        pltpu.make_async_copy(
                            out_v, out_hbm.at[pl.ds(0, G)], sem_o).wait()

                    for r in range(G):
                        i0 = idx_all[pl.ds(ofs + j * IDX_PER_G + r * DEG, L)]
                        i1 = idx_all[pl.ds(ofs + j * IDX_PER_G + r * DEG + L,
                                           L)]
                        nz_s = jnp.sum(jnp.where(i0 == 0, 1.0, 0.0)
                                       + jnp.where(i1 == 0, 1.0, 0.0))
                        nzf = jnp.full((L,), nz_s, dtype=jnp.float32)

                        def body(n, acc):
                            row = r * DEG + n
                            return tuple(
                                acc[v] + rows_v[row, pl.ds(v * L, L)]
                                for v in range(NV))

                        acc = lax.fori_loop(
                            0, DEG, body,
                            tuple(jnp.zeros((L,), jnp.float32)
                                  for _ in range(NV)))
                        cnt = jnp.float32(DEG) - nzf
                        cnt = jnp.where(cnt == 0.0, 1.0, cnt)
                        scale = 1.0 / cnt
                        for v in range(NV):
                            out_v[r, pl.ds(v * L, L)] = (
                                acc[v] - nzf * row0_v[pl.ds(v * L, L)]) * scale

                    pltpu.async_copy(
                        out_v, out_hbm.at[pl.ds((g0 + j) * G, G)], sem_o)
                    start_fetch(j + NBUF, rows_v, sem)

            return carry

        lax.fori_loop(0, K_PAD // NBUF, step, 0)

        for b in range(NBUF):
            pltpu.make_async_copy(
                bufs[b][1], out_hbm.at[pl.ds(0, G)], bufs[b][3]).wait()

    return agg


def kernel(feature_table, nodes, neigh_index, feature_dim):
    del nodes, feature_dim
    neigh_flat = neigh_index.reshape(-1).astype(jnp.int32)
    return _build()(feature_table, neigh_flat)
